# baseline (speedup 1.0000x reference)
import functools

import jax
import jax.numpy as jnp
from jax import lax
from jax.experimental import pallas as pl
from jax.experimental.pallas import tpu as pltpu
from jax.experimental.pallas import tpu_sc as plsc

_INFO = plsc.get_sparse_core_info()
_NC, _NS, _L = _INFO.num_cores, _INFO.num_subcores, _INFO.num_lanes
_NW = _NC * _NS
_C = 16
_NXB = 2


def _sc_add(x2d, pe, pos, *, b, t):
    n, d = x2d.shape
    tw = t // _NW
    nchunks = tw // _C
    mesh = plsc.VectorSubcoreMesh(core_axis_name="c", subcore_axis_name="s")

    @functools.partial(
        pl.kernel,
        mesh=mesh,
        out_type=jax.ShapeDtypeStruct((n, d), jnp.float32),
        scratch_types=[
            pltpu.VMEM((tw,), jnp.int32),
            pltpu.VMEM((_C, d), jnp.float32),
            pltpu.VMEM((_NXB, _C, d), jnp.float32),
            pltpu.SemaphoreType.DMA,
            pltpu.SemaphoreType.DMA((_NXB,)),
            pltpu.SemaphoreType.DMA((_NXB,)),
        ],
    )
    def k(x_hbm, pe_hbm, pos_hbm, out_hbm, idx_v, pe_v, x_v, sem_pe,
          sem_ld, sem_st):
        wid = lax.axis_index("s") * _NC + lax.axis_index("c")
        t0 = wid * tw

        def x_load(kk, bb, buf):
            row0 = bb * t + t0 + kk * _C
            return pltpu.async_copy(
                x_hbm.at[pl.ds(row0, _C)], x_v.at[buf], sem_ld.at[buf]
            )

        def x_store(kk, bb, buf):
            row0 = bb * t + t0 + kk * _C
            return pltpu.async_copy(
                x_v.at[buf], out_hbm.at[pl.ds(row0, _C)], sem_st.at[buf]
            )

        pltpu.sync_copy(pos_hbm.at[pl.ds(t0, tw)], idx_v)
        steps = [(kk, bb) for kk in range(nchunks) for bb in range(b)]
        h_ld = {0: x_load(*steps[0], 0)}
        h_st = {}

        for si, (kk, bb) in enumerate(steps):
            buf = si % _NXB
            if bb == 0:
                pltpu.async_copy(
                    pe_hbm.at[idx_v.at[pl.ds(kk * _C, _C)]], pe_v, sem_pe
                ).wait()
            nsi = si + _NXB - 1
            if nsi < len(steps):
                nbuf = nsi % _NXB
                if nbuf in h_st:
                    h_st.pop(nbuf).wait()
                h_ld[nsi] = x_load(*steps[nsi], nbuf)
            h_ld.pop(si).wait()

            def vec(jj, _):
                r = jj // (d // _L)
                col = (jj % (d // _L)) * _L
                x_v[buf, r, pl.ds(col, _L)] = (
                    x_v[buf, r, pl.ds(col, _L)] + pe_v[r, pl.ds(col, _L)]
                )
                return _

            lax.fori_loop(0, _C * (d // _L), vec, None, unroll=8)
            h_st[buf] = x_store(kk, bb, buf)

        for buf in list(h_st):
            h_st.pop(buf).wait()

    return k(x2d, pe, pos)


def kernel(x, pe, offset=0):
    b, t, d = x.shape
    pos = jnp.arange(t, dtype=jnp.int32) + jnp.asarray(offset, jnp.int32)
    out = _sc_add(x.reshape(b * t, d), pe, pos, b=b, t=t)
    return out.reshape(b, t, d)


# speedup vs baseline: 2.8310x; 2.8310x over previous
import functools

import jax
import jax.numpy as jnp
from jax.experimental import pallas as pl
from jax.experimental.pallas import tpu as pltpu


def _body(off_ref, x_ref, pe_hbm, o_ref, pe_buf, sems, *, tb, nt):
    i = pl.program_id(0)
    off = pl.multiple_of(off_ref[0], 8)

    @pl.when(i == 0)
    def _prologue():
        pltpu.make_async_copy(
            pe_hbm.at[pl.ds(off, tb)], pe_buf.at[0], sems.at[0]
        ).start()

    @pl.when(i + 1 < nt)
    def _prefetch_next():
        pltpu.make_async_copy(
            pe_hbm.at[pl.ds(off + (i + 1) * tb, tb)],
            pe_buf.at[(i + 1) % 2],
            sems.at[(i + 1) % 2],
        ).start()

    pltpu.make_async_copy(
        pe_hbm.at[pl.ds(off + i * tb, tb)], pe_buf.at[i % 2], sems.at[i % 2]
    ).wait()

    o_ref[...] = x_ref[...] + pe_buf[i % 2][None, :, :]


@functools.partial(jax.jit, static_argnames=("tb",))
def _lpe_add(x, pe, offset_arr, tb=256):
    b, t, d = x.shape
    nt = t // tb
    body = functools.partial(_body, tb=tb, nt=nt)
    return pl.pallas_call(
        body,
        grid=(nt,),
        in_specs=[
            pl.BlockSpec((1,), lambda i: (0,), memory_space=pltpu.MemorySpace.SMEM),
            pl.BlockSpec((b, tb, d), lambda i: (0, i, 0)),
            pl.BlockSpec(memory_space=pl.ANY),
        ],
        out_specs=pl.BlockSpec((b, tb, d), lambda i: (0, i, 0)),
        out_shape=jax.ShapeDtypeStruct((b, t, d), x.dtype),
        scratch_shapes=[
            pltpu.VMEM((2, tb, d), x.dtype),
            pltpu.SemaphoreType.DMA((2,)),
        ],
    )(offset_arr, x, pe)


def kernel(x, pe, offset=0):
    offset_arr = jnp.asarray(offset, jnp.int32).reshape((1,))
    return _lpe_add(x, pe, offset_arr)
